# Initial kernel scaffold; baseline (speedup 1.0000x reference)
import jax
import jax.numpy as jnp
from jax.experimental import pallas as pl


def kernel(x, edge_index, edge_attr, batch, W1, b1, W2, b2, W3, b3, Wfc, bfc):
    raise NotImplementedError("write your pallas kernel here")


# speedup vs baseline: 6.6796x; 6.6796x over previous
import functools

import jax
import jax.numpy as jnp
from jax import lax
from jax.experimental import pallas as pl
from jax.experimental.pallas import tpu as pltpu
from jax.experimental.pallas import tpu_sc as plsc

N = 10000
E = 160000
G = 16
H0, H1, H2, H3 = 8, 128, 256, 512

NC, NS = 2, 16
NW = NC * NS
CHUNK = 128
NCHUNK = 40
EPT = NCHUNK * CHUNK
EPAD = NW * EPT
NPAD = 10240
RPT = NPAD // NS
ZB = 64
BM = 1024

_mesh = plsc.VectorSubcoreMesh(
    core_axis_name="c", subcore_axis_name="s", num_cores=NC, num_subcores=NS)


def _deg_body(col_hbm, ew_hbm, out_hbm, colbuf, ewbuf, zbuf, acc, sem):
    del sem
    cid = lax.axis_index("c")
    sid = lax.axis_index("s")
    wg = cid * NS + sid
    zero = jnp.zeros((16,), jnp.float32)
    for i in range(RPT // 16):
        zbuf[pl.ds(i * 16, 16)] = zero
    pltpu.sync_copy(zbuf, acc.at[pl.ds(sid * RPT, RPT)])
    plsc.subcore_barrier()
    pltpu.sync_copy(col_hbm.at[wg], colbuf)
    pltpu.sync_copy(ew_hbm.at[wg], ewbuf)

    def chunk(j, carry):
        pltpu.sync_copy(ewbuf.at[j], acc.at[colbuf.at[j]], add=True)
        return carry

    lax.fori_loop(0, NCHUNK, chunk, 0)
    plsc.subcore_barrier()
    pltpu.sync_copy(acc.at[pl.ds(sid * RPT, RPT)],
                    out_hbm.at[cid, pl.ds(sid * RPT, RPT)])


_deg_call = pl.kernel(
    _deg_body,
    out_type=jax.ShapeDtypeStruct((NC, NPAD), jnp.float32),
    mesh=_mesh,
    scratch_types=[
        pltpu.VMEM((NCHUNK, CHUNK), jnp.int32),
        pltpu.VMEM((NCHUNK, CHUNK), jnp.float32),
        pltpu.VMEM((RPT,), jnp.float32),
        pltpu.VMEM_SHARED((NPAD,), jnp.float32),
        pltpu.SemaphoreType.DMA,
    ],
)


def _spmm_body(F, g_hbm, row_hbm, col_hbm, ew_hbm, out_hbm,
               rowbuf, colbuf, ewbuf, rows, zbuf, acc, sem):
    cid = lax.axis_index("c")
    sid = lax.axis_index("s")
    wg = cid * NS + sid
    nk = F // 16
    zero = jnp.zeros((16,), jnp.float32)

    def zrow(i, carry):
        for k in range(nk):
            zbuf[i, pl.ds(k * 16, 16)] = zero
        return carry

    lax.fori_loop(0, ZB, zrow, 0)
    for t in range(RPT // ZB):
        pltpu.sync_copy(zbuf, acc.at[pl.ds(sid * RPT + t * ZB, ZB)])
    plsc.subcore_barrier()

    pltpu.sync_copy(row_hbm.at[wg], rowbuf)
    pltpu.sync_copy(col_hbm.at[wg], colbuf)
    pltpu.sync_copy(ew_hbm.at[wg], ewbuf)

    def chunk(j, carry):
        pltpu.async_copy(g_hbm.at[rowbuf.at[j]], rows, sem).wait()

        def edges(eo, c2):
            wv = ewbuf[j, pl.ds(eo * 16, 16)]
            for u in range(16):
                e = eo * 16 + u
                w = jnp.full((16,), wv[u], jnp.float32)
                for k in range(nk):
                    rows[e, pl.ds(k * 16, 16)] = rows[e, pl.ds(k * 16, 16)] * w
            return c2

        lax.fori_loop(0, CHUNK // 16, edges, 0)
        pltpu.sync_copy(rows, acc.at[colbuf.at[j]], add=True)
        return carry

    lax.fori_loop(0, NCHUNK, chunk, 0)
    plsc.subcore_barrier()
    pltpu.sync_copy(acc.at[pl.ds(sid * RPT, RPT)],
                    out_hbm.at[cid, pl.ds(sid * RPT, RPT)])


def _make_spmm(F):
    return pl.kernel(
        functools.partial(_spmm_body, F),
        out_type=jax.ShapeDtypeStruct((NC, NPAD, F), jnp.float32),
        mesh=_mesh,
        scratch_types=[
            pltpu.VMEM((NCHUNK, CHUNK), jnp.int32),
            pltpu.VMEM((NCHUNK, CHUNK), jnp.int32),
            pltpu.VMEM((NCHUNK, CHUNK), jnp.float32),
            pltpu.VMEM((CHUNK, F), jnp.float32),
            pltpu.VMEM((ZB, F), jnp.float32),
            pltpu.VMEM_SHARED((NPAD, F), jnp.float32),
            pltpu.SemaphoreType.DMA,
        ],
    )


_spmm128 = _make_spmm(128)


def _dinv_body(dp_ref, x_ref, dinv_ref, g1_ref):
    dp = dp_ref[...]
    deg = dp[:, 0:1] + dp[:, 1:2] + 1.0
    dinv = lax.rsqrt(deg)
    dinv_ref[...] = dinv
    g1_ref[...] = x_ref[...] * dinv


def _dinv_call(dp, xp):
    return pl.pallas_call(
        _dinv_body,
        out_shape=(jax.ShapeDtypeStruct((NPAD, 1), jnp.float32),
                   jax.ShapeDtypeStruct((NPAD, 128), jnp.float32)),
    )(dp, xp)


def _mm_body(nchunks, scale_out, refs):
    nin = 3 * nchunks
    dinv = refs[nin][...]
    w = refs[nin + 1][...]
    b = refs[nin + 2][...]
    outs = refs[nin + 3:]
    xs = []
    for c in range(nchunks):
        s0, s1, gc = refs[3 * c], refs[3 * c + 1], refs[3 * c + 2]
        xs.append(s0[...] + s1[...] + gc[...])
    x = jnp.concatenate(xs, axis=1) if nchunks > 1 else xs[0]
    x = x * dinv
    h = jnp.dot(x, w, preferred_element_type=jnp.float32) + b
    h = jnp.maximum(h, 0.0)
    if scale_out:
        h = h * dinv
    no = h.shape[1] // len(outs)
    for c, oref in enumerate(outs):
        oref[...] = h[:, c * no:(c + 1) * no]


def _layer_call(nchunks, nout_splits, K, Nout, scale_out):
    def body(*refs):
        _mm_body(nchunks, scale_out, refs)

    grid = (NPAD // BM,)
    in_specs = []
    for _ in range(nchunks):
        for _ in range(3):
            in_specs.append(pl.BlockSpec((BM, K // nchunks), lambda i: (i, 0)))
    in_specs.append(pl.BlockSpec((BM, 1), lambda i: (i, 0)))
    in_specs.append(pl.BlockSpec((K, Nout), lambda i: (0, 0)))
    in_specs.append(pl.BlockSpec((1, Nout), lambda i: (0, 0)))
    no = Nout // nout_splits
    out_specs = tuple(pl.BlockSpec((BM, no), lambda i: (i, 0))
                      for _ in range(nout_splits))
    out_shape = tuple(jax.ShapeDtypeStruct((NPAD, no), jnp.float32)
                      for _ in range(nout_splits))
    fn = pl.pallas_call(body, grid=grid, in_specs=in_specs,
                        out_specs=out_specs, out_shape=out_shape)

    def call(*args):
        res = fn(*args)
        return res if nout_splits > 1 else res[0]

    return call


_l1 = _layer_call(1, 1, 128, H1, True)
_l2 = _layer_call(1, 2, H1, H2, True)
_l3 = _layer_call(2, 1, H2, H3, False)


def _pool_body(h3_ref, batch_ref, wfc_ref, bfc_ref, out_ref, pooled, counts):
    i = pl.program_id(0)

    @pl.when(i == 0)
    def _init():
        pooled[...] = jnp.zeros_like(pooled)
        counts[...] = jnp.zeros_like(counts)

    bids = batch_ref[0]
    seg = lax.broadcasted_iota(jnp.int32, (G, BM), 0)
    p = (bids == seg).astype(jnp.float32)
    pooled[...] += jnp.dot(p, h3_ref[...], preferred_element_type=jnp.float32)
    counts[...] += jnp.sum(p, axis=1, keepdims=True)

    @pl.when(i == NPAD // BM - 1)
    def _fin():
        mean = pooled[...] / jnp.maximum(counts[...], 1.0)
        logits = jnp.dot(mean, wfc_ref[...],
                         preferred_element_type=jnp.float32) + bfc_ref[...]
        out_ref[...] = jax.nn.sigmoid(logits)


def _pool_call(h3, batchp, wfc, bfc):
    return pl.pallas_call(
        _pool_body,
        grid=(NPAD // BM,),
        in_specs=[
            pl.BlockSpec((BM, H3), lambda i: (i, 0)),
            pl.BlockSpec((1, 1, BM), lambda i: (i, 0, 0)),
            pl.BlockSpec((H3, 1), lambda i: (0, 0)),
            pl.BlockSpec((1, 1), lambda i: (0, 0)),
        ],
        out_specs=pl.BlockSpec((G, 1), lambda i: (0, 0)),
        out_shape=jax.ShapeDtypeStruct((G, 1), jnp.float32),
        scratch_shapes=[pltpu.VMEM((G, H3), jnp.float32),
                        pltpu.VMEM((G, 1), jnp.float32)],
    )(h3, batchp, wfc, bfc)


def kernel(x, edge_index, edge_attr, batch, W1, b1, W2, b2, W3, b3, Wfc, bfc):
    row = edge_index[0].astype(jnp.int32)
    col = edge_index[1].astype(jnp.int32)
    ew = edge_attr.astype(jnp.float32)
    pad = EPAD - E
    rowp = jnp.concatenate([row, jnp.zeros((pad,), jnp.int32)]).reshape(NW, NCHUNK, CHUNK)
    colp = jnp.concatenate([col, jnp.zeros((pad,), jnp.int32)]).reshape(NW, NCHUNK, CHUNK)
    ewp = jnp.concatenate([ew, jnp.zeros((pad,), jnp.float32)]).reshape(NW, NCHUNK, CHUNK)
    xp = jnp.pad(x, ((0, NPAD - N), (0, 128 - H0)))
    batchp = jnp.concatenate(
        [batch.astype(jnp.int32), jnp.full((NPAD - N,), G, jnp.int32)]
    ).reshape(NPAD // BM, 1, BM)
    W1p = jnp.pad(W1, ((0, 128 - H0), (0, 0)))

    degp = _deg_call(colp, ewp)
    dinv, g1 = _dinv_call(degp.T, xp)
    s1 = _spmm128(g1, rowp, colp, ewp)
    g2 = _l1(s1[0], s1[1], g1, dinv, W1p, b1.reshape(1, H1))
    s2 = _spmm128(g2, rowp, colp, ewp)
    g3a, g3b = _l2(s2[0], s2[1], g2, dinv, W2, b2.reshape(1, H2))
    s3a = _spmm128(g3a, rowp, colp, ewp)
    s3b = _spmm128(g3b, rowp, colp, ewp)
    h3 = _l3(s3a[0], s3a[1], g3a, s3b[0], s3b[1], g3b,
             dinv, W3, b3.reshape(1, H3))
    return _pool_call(h3, batchp, Wfc, bfc.reshape(1, 1))


# speedup vs baseline: 19.9685x; 2.9895x over previous
import functools

import jax
import jax.numpy as jnp
from jax import lax
from jax.experimental import pallas as pl
from jax.experimental.pallas import tpu as pltpu
from jax.experimental.pallas import tpu_sc as plsc

N = 10000
E = 160000
G = 16
H0, H1, H2, H3 = 8, 128, 256, 512

NC, NS = 2, 16
NW = NC * NS
CHUNK = 128
NCHUNK = 40
EPT = NCHUNK * CHUNK
EPAD = NW * EPT
NPAD = 10240
RPT = NPAD // NS
ZB = 64
BM = 1024

_mesh = plsc.VectorSubcoreMesh(
    core_axis_name="c", subcore_axis_name="s", num_cores=NC, num_subcores=NS)


def _deg_body(col_hbm, ew_hbm, out_hbm, colbuf, ewbuf, zbuf, acc, sem):
    del sem
    cid = lax.axis_index("c")
    sid = lax.axis_index("s")
    wg = cid * NS + sid
    zero = jnp.zeros((16,), jnp.float32)
    for i in range(RPT // 16):
        zbuf[pl.ds(i * 16, 16)] = zero
    pltpu.sync_copy(zbuf, acc.at[pl.ds(sid * RPT, RPT)])
    plsc.subcore_barrier()
    pltpu.sync_copy(col_hbm.at[wg], colbuf)
    pltpu.sync_copy(ew_hbm.at[wg], ewbuf)

    def chunk(j, carry):
        pltpu.sync_copy(ewbuf.at[j], acc.at[colbuf.at[j]], add=True)
        return carry

    lax.fori_loop(0, NCHUNK, chunk, 0)
    plsc.subcore_barrier()
    pltpu.sync_copy(acc.at[pl.ds(sid * RPT, RPT)],
                    out_hbm.at[cid, pl.ds(sid * RPT, RPT)])


_deg_call = pl.kernel(
    _deg_body,
    out_type=jax.ShapeDtypeStruct((NC, NPAD), jnp.float32),
    mesh=_mesh,
    scratch_types=[
        pltpu.VMEM((NCHUNK, CHUNK), jnp.int32),
        pltpu.VMEM((NCHUNK, CHUNK), jnp.float32),
        pltpu.VMEM((RPT,), jnp.float32),
        pltpu.VMEM_SHARED((NPAD,), jnp.float32),
        pltpu.SemaphoreType.DMA,
    ],
)


def _spmm_body(F, g_hbm, row_hbm, col_hbm, ew_hbm, out_hbm,
               rowbuf, colbuf, ewbuf, rows0, rows1, acc, sem0, sem1):
    cid = lax.axis_index("c")
    sid = lax.axis_index("s")
    wg = cid * NS + sid
    nk = F // 16
    zero = jnp.zeros((16,), jnp.float32)
    rbufs = (rows0, rows1)
    sems = (sem0, sem1)

    def zrow(i, carry):
        for k in range(nk):
            rows0[i, pl.ds(k * 16, 16)] = zero
        return carry

    lax.fori_loop(0, CHUNK, zrow, 0)
    for t in range(RPT // CHUNK):
        pltpu.sync_copy(rows0, acc.at[pl.ds(sid * RPT + t * CHUNK, CHUNK)])
    plsc.subcore_barrier()

    pltpu.sync_copy(row_hbm.at[wg], rowbuf)
    pltpu.sync_copy(col_hbm.at[wg], colbuf)
    pltpu.sync_copy(ew_hbm.at[wg], ewbuf)

    pltpu.async_copy(g_hbm.at[rowbuf.at[0]], rows0, sem0)

    def chunk2(jj, carry):
        for b in range(2):
            j = jj * 2 + b
            nb = 1 - b

            @pl.when(j < NCHUNK - 1)
            def _():
                pltpu.async_copy(g_hbm.at[rowbuf.at[j + 1]], rbufs[nb], sems[nb])

            pltpu.make_async_copy(g_hbm.at[rowbuf.at[j]], rbufs[b], sems[b]).wait()
            rows = rbufs[b]

            def edges(eo, c2):
                wv = ewbuf[j, pl.ds(eo * 16, 16)]
                for u in range(16):
                    e = eo * 16 + u
                    w = jnp.full((16,), wv[u], jnp.float32)
                    for k in range(nk):
                        rows[e, pl.ds(k * 16, 16)] = rows[e, pl.ds(k * 16, 16)] * w
                return c2

            lax.fori_loop(0, CHUNK // 16, edges, 0)

            pltpu.sync_copy(rows, acc.at[colbuf.at[j]], add=True)
        return carry

    lax.fori_loop(0, NCHUNK // 2, chunk2, 0)
    plsc.subcore_barrier()
    pltpu.sync_copy(acc.at[pl.ds(sid * RPT, RPT)],
                    out_hbm.at[cid, pl.ds(sid * RPT, RPT)])


def _make_spmm(F):
    return pl.kernel(
        functools.partial(_spmm_body, F),
        out_type=jax.ShapeDtypeStruct((NC, NPAD, F), jnp.float32),
        mesh=_mesh,
        scratch_types=[
            pltpu.VMEM((NCHUNK, CHUNK), jnp.int32),
            pltpu.VMEM((NCHUNK, CHUNK), jnp.int32),
            pltpu.VMEM((NCHUNK, CHUNK), jnp.float32),
            pltpu.VMEM((CHUNK, F), jnp.float32),
            pltpu.VMEM((CHUNK, F), jnp.float32),
            pltpu.VMEM_SHARED((NPAD, F), jnp.float32),
            pltpu.SemaphoreType.DMA,
            pltpu.SemaphoreType.DMA,
        ],
    )


_spmm128 = _make_spmm(128)


def _dinv_body(dp_ref, x_ref, dinv_ref, g1_ref):
    dp = dp_ref[...]
    deg = dp[:, 0:1] + dp[:, 1:2] + 1.0
    dinv = lax.rsqrt(deg)
    dinv_ref[...] = dinv
    g1_ref[...] = x_ref[...] * dinv


def _dinv_call(dp, xp):
    return pl.pallas_call(
        _dinv_body,
        out_shape=(jax.ShapeDtypeStruct((NPAD, 1), jnp.float32),
                   jax.ShapeDtypeStruct((NPAD, 128), jnp.float32)),
    )(dp, xp)


def _mm_body(nchunks, scale_out, refs):
    nin = 3 * nchunks
    dinv = refs[nin][...]
    w = refs[nin + 1][...]
    b = refs[nin + 2][...]
    outs = refs[nin + 3:]
    xs = []
    for c in range(nchunks):
        s0, s1, gc = refs[3 * c], refs[3 * c + 1], refs[3 * c + 2]
        xs.append(s0[...] + s1[...] + gc[...])
    x = jnp.concatenate(xs, axis=1) if nchunks > 1 else xs[0]
    x = x * dinv
    h = jnp.dot(x, w, preferred_element_type=jnp.float32) + b
    h = jnp.maximum(h, 0.0)
    if scale_out:
        h = h * dinv
    no = h.shape[1] // len(outs)
    for c, oref in enumerate(outs):
        oref[...] = h[:, c * no:(c + 1) * no]


def _layer_call(nchunks, nout_splits, K, Nout, scale_out):
    def body(*refs):
        _mm_body(nchunks, scale_out, refs)

    grid = (NPAD // BM,)
    in_specs = []
    for _ in range(nchunks):
        for _ in range(3):
            in_specs.append(pl.BlockSpec((BM, K // nchunks), lambda i: (i, 0)))
    in_specs.append(pl.BlockSpec((BM, 1), lambda i: (i, 0)))
    in_specs.append(pl.BlockSpec((K, Nout), lambda i: (0, 0)))
    in_specs.append(pl.BlockSpec((1, Nout), lambda i: (0, 0)))
    no = Nout // nout_splits
    out_specs = tuple(pl.BlockSpec((BM, no), lambda i: (i, 0))
                      for _ in range(nout_splits))
    out_shape = tuple(jax.ShapeDtypeStruct((NPAD, no), jnp.float32)
                      for _ in range(nout_splits))
    fn = pl.pallas_call(body, grid=grid, in_specs=in_specs,
                        out_specs=out_specs, out_shape=out_shape)

    def call(*args):
        res = fn(*args)
        return res if nout_splits > 1 else res[0]

    return call


_l1 = _layer_call(1, 1, 128, H1, True)
_l2 = _layer_call(1, 2, H1, H2, True)
_l3 = _layer_call(2, 1, H2, H3, False)


def _pool_body(h3_ref, batch_ref, wfc_ref, bfc_ref, out_ref, pooled, counts):
    i = pl.program_id(0)

    @pl.when(i == 0)
    def _init():
        pooled[...] = jnp.zeros_like(pooled)
        counts[...] = jnp.zeros_like(counts)

    bids = batch_ref[0]
    seg = lax.broadcasted_iota(jnp.int32, (G, BM), 0)
    p = (bids == seg).astype(jnp.float32)
    pooled[...] += jnp.dot(p, h3_ref[...], preferred_element_type=jnp.float32)
    counts[...] += jnp.sum(p, axis=1, keepdims=True)

    @pl.when(i == NPAD // BM - 1)
    def _fin():
        mean = pooled[...] / jnp.maximum(counts[...], 1.0)
        logits = jnp.dot(mean, wfc_ref[...],
                         preferred_element_type=jnp.float32) + bfc_ref[...]
        out_ref[...] = jax.nn.sigmoid(logits)


def _pool_call(h3, batchp, wfc, bfc):
    return pl.pallas_call(
        _pool_body,
        grid=(NPAD // BM,),
        in_specs=[
            pl.BlockSpec((BM, H3), lambda i: (i, 0)),
            pl.BlockSpec((1, 1, BM), lambda i: (i, 0, 0)),
            pl.BlockSpec((H3, 1), lambda i: (0, 0)),
            pl.BlockSpec((1, 1), lambda i: (0, 0)),
        ],
        out_specs=pl.BlockSpec((G, 1), lambda i: (0, 0)),
        out_shape=jax.ShapeDtypeStruct((G, 1), jnp.float32),
        scratch_shapes=[pltpu.VMEM((G, H3), jnp.float32),
                        pltpu.VMEM((G, 1), jnp.float32)],
    )(h3, batchp, wfc, bfc)


def kernel(x, edge_index, edge_attr, batch, W1, b1, W2, b2, W3, b3, Wfc, bfc):
    row = edge_index[0].astype(jnp.int32)
    col = edge_index[1].astype(jnp.int32)
    ew = edge_attr.astype(jnp.float32)
    pad = EPAD - E
    padids = (N + jnp.arange(pad, dtype=jnp.int32) % (NPAD - N)).astype(jnp.int32)
    rowp = jnp.concatenate([row, padids]).reshape(NW, NCHUNK, CHUNK)
    colp = jnp.concatenate([col, padids]).reshape(NW, NCHUNK, CHUNK)
    ewp = jnp.concatenate([ew, jnp.zeros((pad,), jnp.float32)]).reshape(NW, NCHUNK, CHUNK)
    xp = jnp.pad(x, ((0, NPAD - N), (0, 128 - H0)))
    batchp = jnp.concatenate(
        [batch.astype(jnp.int32), jnp.full((NPAD - N,), G, jnp.int32)]
    ).reshape(NPAD // BM, 1, BM)
    W1p = jnp.pad(W1, ((0, 128 - H0), (0, 0)))

    degp = _deg_call(colp, ewp)
    dinv, g1 = _dinv_call(degp.T, xp)
    s1 = _spmm128(g1, rowp, colp, ewp)
    g2 = _l1(s1[0], s1[1], g1, dinv, W1p, b1.reshape(1, H1))
    s2 = _spmm128(g2, rowp, colp, ewp)
    g3a, g3b = _l2(s2[0], s2[1], g2, dinv, W2, b2.reshape(1, H2))
    s3a = _spmm128(g3a, rowp, colp, ewp)
    s3b = _spmm128(g3b, rowp, colp, ewp)
    h3 = _l3(s3a[0], s3a[1], g3a, s3b[0], s3b[1], g3b,
             dinv, W3, b3.reshape(1, H3))
    return _pool_call(h3, batchp, Wfc, bfc.reshape(1, 1))


# speedup vs baseline: 20.3801x; 1.0206x over previous
import functools

import jax
import jax.numpy as jnp
from jax import lax
from jax.experimental import pallas as pl
from jax.experimental.pallas import tpu as pltpu
from jax.experimental.pallas import tpu_sc as plsc

N = 10000
E = 160000
G = 16
H0, H1, H2, H3 = 8, 128, 256, 512

NC, NS = 2, 16
NW = NC * NS
CHUNK = 64
NCHUNK = 81
EPT = NCHUNK * CHUNK
EPAD = NW * EPT
NPAD = 10240
RPT = NPAD // NS
BM = 1024

_mesh = plsc.VectorSubcoreMesh(
    core_axis_name="c", subcore_axis_name="s", num_cores=NC, num_subcores=NS)


def _deg_body(col_hbm, ew_hbm, out_hbm, colbuf, ewbuf, zbuf, acc, sem):
    del sem
    cid = lax.axis_index("c")
    sid = lax.axis_index("s")
    wg = cid * NS + sid
    zero = jnp.zeros((16,), jnp.float32)
    for i in range(RPT // 16):
        zbuf[pl.ds(i * 16, 16)] = zero
    pltpu.sync_copy(zbuf, acc.at[pl.ds(sid * RPT, RPT)])
    plsc.subcore_barrier()
    pltpu.sync_copy(col_hbm.at[wg], colbuf)
    pltpu.sync_copy(ew_hbm.at[wg], ewbuf)

    def chunk(j, carry):
        pltpu.sync_copy(ewbuf.at[j], acc.at[colbuf.at[j]], add=True)
        return carry

    lax.fori_loop(0, NCHUNK, chunk, 0)
    plsc.subcore_barrier()
    pltpu.sync_copy(acc.at[pl.ds(sid * RPT, RPT)],
                    out_hbm.at[cid, pl.ds(sid * RPT, RPT)])


_deg_call = pl.kernel(
    _deg_body,
    out_type=jax.ShapeDtypeStruct((NC, NPAD), jnp.float32),
    mesh=_mesh,
    scratch_types=[
        pltpu.VMEM((NCHUNK, CHUNK), jnp.int32),
        pltpu.VMEM((NCHUNK, CHUNK), jnp.float32),
        pltpu.VMEM((RPT,), jnp.float32),
        pltpu.VMEM_SHARED((NPAD,), jnp.float32),
        pltpu.SemaphoreType.DMA,
    ],
)


def _spmm_body(F, g_hbm, rowew_hbm, col_hbm, out_hbm,
               rowew, colbuf, rows0, rows1, rows2, acc,
               gsem0, gsem1, gsem2, ssem0, ssem1, ssem2):
    cid = lax.axis_index("c")
    sid = lax.axis_index("s")
    wg = cid * NS + sid
    nk = F // 16
    zero = jnp.zeros((16,), jnp.float32)
    rbufs = (rows0, rows1, rows2)
    gsems = (gsem0, gsem1, gsem2)
    ssems = (ssem0, ssem1, ssem2)

    def zrow(i, carry):
        for k in range(nk):
            rows0[i, pl.ds(k * 16, 16)] = zero
        return carry

    lax.fori_loop(0, CHUNK, zrow, 0)
    for t in range(RPT // CHUNK):
        pltpu.sync_copy(rows0, acc.at[pl.ds(sid * RPT + t * CHUNK, CHUNK)])
    plsc.subcore_barrier()

    pltpu.sync_copy(rowew_hbm.at[wg], rowew)
    pltpu.sync_copy(col_hbm.at[wg], colbuf)

    pltpu.async_copy(g_hbm.at[rowew.at[0, pl.ds(0, CHUNK)]], rows0, gsem0)
    pltpu.async_copy(g_hbm.at[rowew.at[1, pl.ds(0, CHUNK)]], rows1, gsem1)

    def chunk3(jj, carry):
        for b in range(3):
            j = jj * 3 + b
            bp = (b + 2) % 3
            rows = rbufs[b]

            pltpu.make_async_copy(g_hbm.at[rowew.at[j, pl.ds(0, CHUNK)]],
                                  rows, gsems[b]).wait()

            @plsc.parallel_loop(0, CHUNK // 16)
            def edges(eo):
                wv = lax.bitcast_convert_type(
                    rowew[j, pl.ds(CHUNK + eo * 16, 16)], jnp.float32)
                for u in range(16):
                    e = eo * 16 + u
                    w = jnp.full((16,), wv[u], jnp.float32)
                    for k in range(nk):
                        rows[e, pl.ds(k * 16, 16)] = rows[e, pl.ds(k * 16, 16)] * w

            pltpu.async_copy(rows, acc.at[colbuf.at[j]], ssems[b], add=True)

            @pl.when(j >= 1)
            def _():
                pltpu.make_async_copy(rbufs[bp], acc.at[colbuf.at[j - 1]],
                                      ssems[bp]).wait()

            @pl.when(j + 2 < NCHUNK)
            def _():
                pltpu.async_copy(g_hbm.at[rowew.at[j + 2, pl.ds(0, CHUNK)]],
                                 rbufs[bp], gsems[bp])
        return carry

    lax.fori_loop(0, NCHUNK // 3, chunk3, 0)
    last = (NCHUNK - 1) % 3
    pltpu.make_async_copy(rbufs[last], acc.at[colbuf.at[NCHUNK - 1]],
                          ssems[last]).wait()
    plsc.subcore_barrier()
    pltpu.sync_copy(acc.at[pl.ds(sid * RPT, RPT)],
                    out_hbm.at[cid, pl.ds(sid * RPT, RPT)])


def _make_spmm(F):
    return pl.kernel(
        functools.partial(_spmm_body, F),
        out_type=jax.ShapeDtypeStruct((NC, NPAD, F), jnp.float32),
        mesh=_mesh,
        scratch_types=[
            pltpu.VMEM((NCHUNK, 2 * CHUNK), jnp.int32),
            pltpu.VMEM((NCHUNK, CHUNK), jnp.int32),
            pltpu.VMEM((CHUNK, F), jnp.float32),
            pltpu.VMEM((CHUNK, F), jnp.float32),
            pltpu.VMEM((CHUNK, F), jnp.float32),
            pltpu.VMEM_SHARED((NPAD, F), jnp.float32),
            pltpu.SemaphoreType.DMA,
            pltpu.SemaphoreType.DMA,
            pltpu.SemaphoreType.DMA,
            pltpu.SemaphoreType.DMA,
            pltpu.SemaphoreType.DMA,
            pltpu.SemaphoreType.DMA,
        ],
    )


_spmm128 = _make_spmm(128)


def _dinv_body(dp_ref, x_ref, dinv_ref, g1_ref):
    dp = dp_ref[...]
    deg = dp[:, 0:1] + dp[:, 1:2] + 1.0
    dinv = lax.rsqrt(deg)
    dinv_ref[...] = dinv
    g1_ref[...] = x_ref[...] * dinv


def _dinv_call(dp, xp):
    return pl.pallas_call(
        _dinv_body,
        out_shape=(jax.ShapeDtypeStruct((NPAD, 1), jnp.float32),
                   jax.ShapeDtypeStruct((NPAD, 128), jnp.float32)),
    )(dp, xp)


def _mm_body(nchunks, scale_out, refs):
    nin = 3 * nchunks
    dinv = refs[nin][...]
    w = refs[nin + 1][...]
    b = refs[nin + 2][...]
    outs = refs[nin + 3:]
    xs = []
    for c in range(nchunks):
        s0, s1, gc = refs[3 * c], refs[3 * c + 1], refs[3 * c + 2]
        xs.append(s0[...] + s1[...] + gc[...])
    x = jnp.concatenate(xs, axis=1) if nchunks > 1 else xs[0]
    x = x * dinv
    h = jnp.dot(x, w, preferred_element_type=jnp.float32) + b
    h = jnp.maximum(h, 0.0)
    if scale_out:
        h = h * dinv
    no = h.shape[1] // len(outs)
    for c, oref in enumerate(outs):
        oref[...] = h[:, c * no:(c + 1) * no]


def _layer_call(nchunks, nout_splits, K, Nout, scale_out):
    def body(*refs):
        _mm_body(nchunks, scale_out, refs)

    grid = (NPAD // BM,)
    in_specs = []
    for _ in range(nchunks):
        for _ in range(3):
            in_specs.append(pl.BlockSpec((BM, K // nchunks), lambda i: (i, 0)))
    in_specs.append(pl.BlockSpec((BM, 1), lambda i: (i, 0)))
    in_specs.append(pl.BlockSpec((K, Nout), lambda i: (0, 0)))
    in_specs.append(pl.BlockSpec((1, Nout), lambda i: (0, 0)))
    no = Nout // nout_splits
    out_specs = tuple(pl.BlockSpec((BM, no), lambda i: (i, 0))
                      for _ in range(nout_splits))
    out_shape = tuple(jax.ShapeDtypeStruct((NPAD, no), jnp.float32)
                      for _ in range(nout_splits))
    fn = pl.pallas_call(body, grid=grid, in_specs=in_specs,
                        out_specs=out_specs, out_shape=out_shape)

    def call(*args):
        res = fn(*args)
        return res if nout_splits > 1 else res[0]

    return call


_l1 = _layer_call(1, 1, 128, H1, True)
_l2 = _layer_call(1, 2, H1, H2, True)


def _l3pool_body(s0a, s1a, ga, s0b, s1b, gb, dinv_ref, w_ref, b_ref,
                 batch_ref, wfc_ref, bfc_ref, out_ref, pooled, counts):
    i = pl.program_id(0)

    @pl.when(i == 0)
    def _init():
        pooled[...] = jnp.zeros_like(pooled)
        counts[...] = jnp.zeros_like(counts)

    dinv = dinv_ref[...]
    x = jnp.concatenate(
        [s0a[...] + s1a[...] + ga[...], s0b[...] + s1b[...] + gb[...]], axis=1)
    x = x * dinv
    h = jnp.dot(x, w_ref[...], preferred_element_type=jnp.float32) + b_ref[...]
    h = jnp.maximum(h, 0.0)

    bids = batch_ref[0]
    seg = lax.broadcasted_iota(jnp.int32, (G, BM), 0)
    p = (bids == seg).astype(jnp.float32)
    pooled[...] += jnp.dot(p, h, preferred_element_type=jnp.float32)
    counts[...] += jnp.sum(p, axis=1, keepdims=True)

    @pl.when(i == NPAD // BM - 1)
    def _fin():
        mean = pooled[...] / jnp.maximum(counts[...], 1.0)
        logits = jnp.dot(mean, wfc_ref[...],
                         preferred_element_type=jnp.float32) + bfc_ref[...]
        out_ref[...] = jax.nn.sigmoid(logits)


def _l3pool_call(s3a0, s3a1, g3a, s3b0, s3b1, g3b, dinv, w3, b3,
                 batchp, wfc, bfc):
    in_specs = [pl.BlockSpec((BM, H1), lambda i: (i, 0)) for _ in range(6)]
    in_specs += [
        pl.BlockSpec((BM, 1), lambda i: (i, 0)),
        pl.BlockSpec((H2, H3), lambda i: (0, 0)),
        pl.BlockSpec((1, H3), lambda i: (0, 0)),
        pl.BlockSpec((1, 1, BM), lambda i: (i, 0, 0)),
        pl.BlockSpec((H3, 1), lambda i: (0, 0)),
        pl.BlockSpec((1, 1), lambda i: (0, 0)),
    ]
    return pl.pallas_call(
        _l3pool_body,
        grid=(NPAD // BM,),
        in_specs=in_specs,
        out_specs=pl.BlockSpec((G, 1), lambda i: (0, 0)),
        out_shape=jax.ShapeDtypeStruct((G, 1), jnp.float32),
        scratch_shapes=[pltpu.VMEM((G, H3), jnp.float32),
                        pltpu.VMEM((G, 1), jnp.float32)],
    )(s3a0, s3a1, g3a, s3b0, s3b1, g3b, dinv, w3, b3, batchp, wfc, bfc)


def kernel(x, edge_index, edge_attr, batch, W1, b1, W2, b2, W3, b3, Wfc, bfc):
    row = edge_index[0].astype(jnp.int32)
    col = edge_index[1].astype(jnp.int32)
    ew = edge_attr.astype(jnp.float32)
    pad = EPAD - E
    padids = (N + jnp.arange(pad, dtype=jnp.int32) % (NPAD - N)).astype(jnp.int32)
    rowp = jnp.concatenate([row, padids]).reshape(NW, NCHUNK, CHUNK)
    colp = jnp.concatenate([col, padids]).reshape(NW, NCHUNK, CHUNK)
    ewp = jnp.concatenate([ew, jnp.zeros((pad,), jnp.float32)]).reshape(NW, NCHUNK, CHUNK)
    rowew = jnp.concatenate(
        [rowp, jax.lax.bitcast_convert_type(ewp, jnp.int32)], axis=-1)
    xp = jnp.pad(x, ((0, NPAD - N), (0, 128 - H0)))
    batchp = jnp.concatenate(
        [batch.astype(jnp.int32), jnp.full((NPAD - N,), G, jnp.int32)]
    ).reshape(NPAD // BM, 1, BM)
    W1p = jnp.pad(W1, ((0, 128 - H0), (0, 0)))

    degp = _deg_call(colp, ewp)
    dinv, g1 = _dinv_call(degp.T, xp)
    s1 = _spmm128(g1, rowew, colp)
    g2 = _l1(s1[0], s1[1], g1, dinv, W1p, b1.reshape(1, H1))
    s2 = _spmm128(g2, rowew, colp)
    g3a, g3b = _l2(s2[0], s2[1], g2, dinv, W2, b2.reshape(1, H2))
    s3a = _spmm128(g3a, rowew, colp)
    s3b = _spmm128(g3b, rowew, colp)
    return _l3pool_call(s3a[0], s3a[1], g3a, s3b[0], s3b[1], g3b,
                        dinv, W3, b3.reshape(1, H3),
                        batchp, Wfc, bfc.reshape(1, 1))


# speedup vs baseline: 20.7172x; 1.0165x over previous
import functools

import jax
import jax.numpy as jnp
from jax import lax
from jax.experimental import pallas as pl
from jax.experimental.pallas import tpu as pltpu
from jax.experimental.pallas import tpu_sc as plsc

N = 10000
E = 160000
G = 16
H0, H1, H2, H3 = 8, 128, 256, 512

NC, NS = 2, 16
NW = NC * NS
CHUNK = 64
NCHUNK = 81
EPT = NCHUNK * CHUNK
EPAD = NW * EPT
NPAD = 10240
RPT = NPAD // NS
BM = 1024

_mesh = plsc.VectorSubcoreMesh(
    core_axis_name="c", subcore_axis_name="s", num_cores=NC, num_subcores=NS)


def _deg_body(col_hbm, ew_hbm, out_hbm, colbuf, ewbuf, zbuf, acc, sem):
    del sem
    cid = lax.axis_index("c")
    sid = lax.axis_index("s")
    wg = cid * NS + sid
    zero = jnp.zeros((16,), jnp.float32)
    for i in range(RPT // 16):
        zbuf[pl.ds(i * 16, 16)] = zero
    pltpu.sync_copy(zbuf, acc.at[pl.ds(sid * RPT, RPT)])
    plsc.subcore_barrier()
    pltpu.sync_copy(col_hbm.at[wg], colbuf)
    pltpu.sync_copy(ew_hbm.at[wg], ewbuf)

    def chunk(j, carry):
        pltpu.sync_copy(ewbuf.at[j], acc.at[colbuf.at[j]], add=True)
        return carry

    lax.fori_loop(0, NCHUNK, chunk, 0)
    plsc.subcore_barrier()
    pltpu.sync_copy(acc.at[pl.ds(sid * RPT, RPT)],
                    out_hbm.at[cid, pl.ds(sid * RPT, RPT)])


_deg_call = pl.kernel(
    _deg_body,
    out_type=jax.ShapeDtypeStruct((NC, NPAD), jnp.float32),
    mesh=_mesh,
    scratch_types=[
        pltpu.VMEM((NCHUNK, CHUNK), jnp.int32),
        pltpu.VMEM((NCHUNK, CHUNK), jnp.float32),
        pltpu.VMEM((RPT,), jnp.float32),
        pltpu.VMEM_SHARED((NPAD,), jnp.float32),
        pltpu.SemaphoreType.DMA,
    ],
)


def _spmm_body(F, g_hbm, rowew_hbm, col_hbm, out_hbm,
               rowew, colbuf, rows0, rows1, rows2, acc,
               gsem0, gsem1, gsem2, ssem0, ssem1, ssem2):
    cid = lax.axis_index("c")
    sid = lax.axis_index("s")
    wg = cid * NS + sid
    nk = F // 16
    zero = jnp.zeros((16,), jnp.float32)
    rbufs = (rows0, rows1, rows2)
    gsems = (gsem0, gsem1, gsem2)
    ssems = (ssem0, ssem1, ssem2)

    def zrow(i, carry):
        for k in range(nk):
            rows0[i, pl.ds(k * 16, 16)] = zero
        return carry

    lax.fori_loop(0, CHUNK, zrow, 0)
    for t in range(RPT // CHUNK):
        pltpu.sync_copy(rows0, acc.at[pl.ds(sid * RPT + t * CHUNK, CHUNK)])
    plsc.subcore_barrier()

    pltpu.sync_copy(rowew_hbm.at[wg], rowew)
    pltpu.sync_copy(col_hbm.at[wg], colbuf)

    pltpu.async_copy(g_hbm.at[rowew.at[0, pl.ds(0, CHUNK)]], rows0, gsem0)
    pltpu.async_copy(g_hbm.at[rowew.at[1, pl.ds(0, CHUNK)]], rows1, gsem1)

    def chunk3(jj, carry):
        for b in range(3):
            j = jj * 3 + b
            bp = (b + 2) % 3
            rows = rbufs[b]

            pltpu.make_async_copy(g_hbm.at[rowew.at[j, pl.ds(0, CHUNK)]],
                                  rows, gsems[b]).wait()

            @plsc.parallel_loop(0, CHUNK // 16)
            def edges(eo):
                wv = lax.bitcast_convert_type(
                    rowew[j, pl.ds(CHUNK + eo * 16, 16)], jnp.float32)
                for u in range(16):
                    e = eo * 16 + u
                    w = jnp.full((16,), wv[u], jnp.float32)
                    for k in range(nk):
                        rows[e, pl.ds(k * 16, 16)] = rows[e, pl.ds(k * 16, 16)] * w

            pltpu.async_copy(rows, acc.at[colbuf.at[j]], ssems[b], add=True)

            @pl.when(j >= 1)
            def _():
                pltpu.make_async_copy(rbufs[bp], acc.at[colbuf.at[j - 1]],
                                      ssems[bp]).wait()

            @pl.when(j + 2 < NCHUNK)
            def _():
                pltpu.async_copy(g_hbm.at[rowew.at[j + 2, pl.ds(0, CHUNK)]],
                                 rbufs[bp], gsems[bp])
        return carry

    lax.fori_loop(0, NCHUNK // 3, chunk3, 0)
    last = (NCHUNK - 1) % 3
    pltpu.make_async_copy(rbufs[last], acc.at[colbuf.at[NCHUNK - 1]],
                          ssems[last]).wait()
    plsc.subcore_barrier()
    pltpu.sync_copy(acc.at[pl.ds(sid * RPT, RPT)],
                    out_hbm.at[cid, pl.ds(sid * RPT, RPT)])


def _make_spmm(F):
    return pl.kernel(
        functools.partial(_spmm_body, F),
        out_type=jax.ShapeDtypeStruct((NC, NPAD, F), jnp.float32),
        mesh=_mesh,
        scratch_types=[
            pltpu.VMEM((NCHUNK, 2 * CHUNK), jnp.int32),
            pltpu.VMEM((NCHUNK, CHUNK), jnp.int32),
            pltpu.VMEM((CHUNK, F), jnp.float32),
            pltpu.VMEM((CHUNK, F), jnp.float32),
            pltpu.VMEM((CHUNK, F), jnp.float32),
            pltpu.VMEM_SHARED((NPAD, F), jnp.float32),
            pltpu.SemaphoreType.DMA,
            pltpu.SemaphoreType.DMA,
            pltpu.SemaphoreType.DMA,
            pltpu.SemaphoreType.DMA,
            pltpu.SemaphoreType.DMA,
            pltpu.SemaphoreType.DMA,
        ],
    )


_spmm128 = _make_spmm(128)


def _spmm2_body(ga_hbm, gb_hbm, rowew_hbm, col_hbm, out_hbm,
                rowew, colbuf, rows0, rows1, rows2, acc,
                gsem0, gsem1, gsem2, ssem0, ssem1, ssem2):
    cid = lax.axis_index("c")
    sid = lax.axis_index("s")
    wg = cid * NS + sid
    nk = 8
    zero = jnp.zeros((16,), jnp.float32)
    rbufs = (rows0, rows1, rows2)
    gsems = (gsem0, gsem1, gsem2)
    ssems = (ssem0, ssem1, ssem2)

    pltpu.sync_copy(rowew_hbm.at[wg], rowew)
    pltpu.sync_copy(col_hbm.at[wg], colbuf)

    for q, g_hbm in enumerate((ga_hbm, gb_hbm)):
        def zrow(i, carry):
            for k in range(nk):
                rows0[i, pl.ds(k * 16, 16)] = zero
            return carry

        lax.fori_loop(0, CHUNK, zrow, 0)
        for t in range(RPT // CHUNK):
            pltpu.sync_copy(rows0, acc.at[pl.ds(sid * RPT + t * CHUNK, CHUNK)])
        plsc.subcore_barrier()

        pltpu.async_copy(g_hbm.at[rowew.at[0, pl.ds(0, CHUNK)]], rows0, gsem0)
        pltpu.async_copy(g_hbm.at[rowew.at[1, pl.ds(0, CHUNK)]], rows1, gsem1)

        def chunk3(jj, carry):
            for b in range(3):
                j = jj * 3 + b
                bp = (b + 2) % 3
                rows = rbufs[b]

                pltpu.make_async_copy(g_hbm.at[rowew.at[j, pl.ds(0, CHUNK)]],
                                      rows, gsems[b]).wait()

                @plsc.parallel_loop(0, CHUNK // 16)
                def edges(eo):
                    wv = lax.bitcast_convert_type(
                        rowew[j, pl.ds(CHUNK + eo * 16, 16)], jnp.float32)
                    for u in range(16):
                        e = eo * 16 + u
                        w = jnp.full((16,), wv[u], jnp.float32)
                        for k in range(nk):
                            rows[e, pl.ds(k * 16, 16)] = (
                                rows[e, pl.ds(k * 16, 16)] * w)

                pltpu.async_copy(rows, acc.at[colbuf.at[j]], ssems[b], add=True)

                @pl.when(j >= 1)
                def _():
                    pltpu.make_async_copy(rbufs[bp], acc.at[colbuf.at[j - 1]],
                                          ssems[bp]).wait()

                @pl.when(j + 2 < NCHUNK)
                def _():
                    pltpu.async_copy(g_hbm.at[rowew.at[j + 2, pl.ds(0, CHUNK)]],
                                     rbufs[bp], gsems[bp])
            return carry

        lax.fori_loop(0, NCHUNK // 3, chunk3, 0)
        last = (NCHUNK - 1) % 3
        pltpu.make_async_copy(rbufs[last], acc.at[colbuf.at[NCHUNK - 1]],
                              ssems[last]).wait()
        plsc.subcore_barrier()
        pltpu.sync_copy(acc.at[pl.ds(sid * RPT, RPT)],
                        out_hbm.at[q, cid, pl.ds(sid * RPT, RPT)])
        plsc.subcore_barrier()


_spmm2 = pl.kernel(
    _spmm2_body,
    out_type=jax.ShapeDtypeStruct((2, NC, NPAD, 128), jnp.float32),
    mesh=_mesh,
    scratch_types=[
        pltpu.VMEM((NCHUNK, 2 * CHUNK), jnp.int32),
        pltpu.VMEM((NCHUNK, CHUNK), jnp.int32),
        pltpu.VMEM((CHUNK, 128), jnp.float32),
        pltpu.VMEM((CHUNK, 128), jnp.float32),
        pltpu.VMEM((CHUNK, 128), jnp.float32),
        pltpu.VMEM_SHARED((NPAD, 128), jnp.float32),
        pltpu.SemaphoreType.DMA,
        pltpu.SemaphoreType.DMA,
        pltpu.SemaphoreType.DMA,
        pltpu.SemaphoreType.DMA,
        pltpu.SemaphoreType.DMA,
        pltpu.SemaphoreType.DMA,
    ],
)


def _dinv_body(dp_ref, x_ref, dinv_ref, g1_ref):
    dp = dp_ref[...]
    deg = dp[:, 0:1] + dp[:, 1:2] + 1.0
    dinv = lax.rsqrt(deg)
    dinv_ref[...] = dinv
    g1_ref[...] = x_ref[...] * dinv


def _dinv_call(dp, xp):
    return pl.pallas_call(
        _dinv_body,
        out_shape=(jax.ShapeDtypeStruct((NPAD, 1), jnp.float32),
                   jax.ShapeDtypeStruct((NPAD, 128), jnp.float32)),
    )(dp, xp)


def _mm_body(nchunks, scale_out, refs):
    nin = 3 * nchunks
    dinv = refs[nin][...]
    w = refs[nin + 1][...]
    b = refs[nin + 2][...]
    outs = refs[nin + 3:]
    xs = []
    for c in range(nchunks):
        s0, s1, gc = refs[3 * c], refs[3 * c + 1], refs[3 * c + 2]
        xs.append(s0[...] + s1[...] + gc[...])
    x = jnp.concatenate(xs, axis=1) if nchunks > 1 else xs[0]
    x = x * dinv
    h = jnp.dot(x, w, preferred_element_type=jnp.float32) + b
    h = jnp.maximum(h, 0.0)
    if scale_out:
        h = h * dinv
    no = h.shape[1] // len(outs)
    for c, oref in enumerate(outs):
        oref[...] = h[:, c * no:(c + 1) * no]


def _layer_call(nchunks, nout_splits, K, Nout, scale_out):
    def body(*refs):
        _mm_body(nchunks, scale_out, refs)

    grid = (NPAD // BM,)
    in_specs = []
    for _ in range(nchunks):
        for _ in range(3):
            in_specs.append(pl.BlockSpec((BM, K // nchunks), lambda i: (i, 0)))
    in_specs.append(pl.BlockSpec((BM, 1), lambda i: (i, 0)))
    in_specs.append(pl.BlockSpec((K, Nout), lambda i: (0, 0)))
    in_specs.append(pl.BlockSpec((1, Nout), lambda i: (0, 0)))
    no = Nout // nout_splits
    out_specs = tuple(pl.BlockSpec((BM, no), lambda i: (i, 0))
                      for _ in range(nout_splits))
    out_shape = tuple(jax.ShapeDtypeStruct((NPAD, no), jnp.float32)
                      for _ in range(nout_splits))
    fn = pl.pallas_call(body, grid=grid, in_specs=in_specs,
                        out_specs=out_specs, out_shape=out_shape)

    def call(*args):
        res = fn(*args)
        return res if nout_splits > 1 else res[0]

    return call


_l1 = _layer_call(1, 1, 128, H1, True)
_l2 = _layer_call(1, 2, H1, H2, True)


def _l3pool_body(s0a, s1a, ga, s0b, s1b, gb, dinv_ref, w_ref, b_ref,
                 batch_ref, wfc_ref, bfc_ref, out_ref, pooled, counts):
    i = pl.program_id(0)

    @pl.when(i == 0)
    def _init():
        pooled[...] = jnp.zeros_like(pooled)
        counts[...] = jnp.zeros_like(counts)

    dinv = dinv_ref[...]
    x = jnp.concatenate(
        [s0a[...] + s1a[...] + ga[...], s0b[...] + s1b[...] + gb[...]], axis=1)
    x = x * dinv
    h = jnp.dot(x, w_ref[...], preferred_element_type=jnp.float32) + b_ref[...]
    h = jnp.maximum(h, 0.0)

    bids = batch_ref[0]
    seg = lax.broadcasted_iota(jnp.int32, (G, BM), 0)
    p = (bids == seg).astype(jnp.float32)
    pooled[...] += jnp.dot(p, h, preferred_element_type=jnp.float32)
    counts[...] += jnp.sum(p, axis=1, keepdims=True)

    @pl.when(i == NPAD // BM - 1)
    def _fin():
        mean = pooled[...] / jnp.maximum(counts[...], 1.0)
        logits = jnp.dot(mean, wfc_ref[...],
                         preferred_element_type=jnp.float32) + bfc_ref[...]
        out_ref[...] = jax.nn.sigmoid(logits)


def _l3pool_call(s3a0, s3a1, g3a, s3b0, s3b1, g3b, dinv, w3, b3,
                 batchp, wfc, bfc):
    in_specs = [pl.BlockSpec((BM, H1), lambda i: (i, 0)) for _ in range(6)]
    in_specs += [
        pl.BlockSpec((BM, 1), lambda i: (i, 0)),
        pl.BlockSpec((H2, H3), lambda i: (0, 0)),
        pl.BlockSpec((1, H3), lambda i: (0, 0)),
        pl.BlockSpec((1, 1, BM), lambda i: (i, 0, 0)),
        pl.BlockSpec((H3, 1), lambda i: (0, 0)),
        pl.BlockSpec((1, 1), lambda i: (0, 0)),
    ]
    return pl.pallas_call(
        _l3pool_body,
        grid=(NPAD // BM,),
        in_specs=in_specs,
        out_specs=pl.BlockSpec((G, 1), lambda i: (0, 0)),
        out_shape=jax.ShapeDtypeStruct((G, 1), jnp.float32),
        scratch_shapes=[pltpu.VMEM((G, H3), jnp.float32),
                        pltpu.VMEM((G, 1), jnp.float32)],
    )(s3a0, s3a1, g3a, s3b0, s3b1, g3b, dinv, w3, b3, batchp, wfc, bfc)


def kernel(x, edge_index, edge_attr, batch, W1, b1, W2, b2, W3, b3, Wfc, bfc):
    row = edge_index[0].astype(jnp.int32)
    col = edge_index[1].astype(jnp.int32)
    ew = edge_attr.astype(jnp.float32)
    pad = EPAD - E
    padids = (N + jnp.arange(pad, dtype=jnp.int32) % (NPAD - N)).astype(jnp.int32)
    rowp = jnp.concatenate([row, padids]).reshape(NW, NCHUNK, CHUNK)
    colp = jnp.concatenate([col, padids]).reshape(NW, NCHUNK, CHUNK)
    ewp = jnp.concatenate([ew, jnp.zeros((pad,), jnp.float32)]).reshape(NW, NCHUNK, CHUNK)
    rowew = jnp.concatenate(
        [rowp, jax.lax.bitcast_convert_type(ewp, jnp.int32)], axis=-1)
    xp = jnp.pad(x, ((0, NPAD - N), (0, 128 - H0)))
    batchp = jnp.concatenate(
        [batch.astype(jnp.int32), jnp.full((NPAD - N,), G, jnp.int32)]
    ).reshape(NPAD // BM, 1, BM)
    W1p = jnp.pad(W1, ((0, 128 - H0), (0, 0)))

    degp = _deg_call(colp, ewp)
    dinv, g1 = _dinv_call(degp.T, xp)
    s1 = _spmm128(g1, rowew, colp)
    g2 = _l1(s1[0], s1[1], g1, dinv, W1p, b1.reshape(1, H1))
    s2 = _spmm128(g2, rowew, colp)
    g3a, g3b = _l2(s2[0], s2[1], g2, dinv, W2, b2.reshape(1, H2))
    s3 = _spmm2(g3a, g3b, rowew, colp)
    return _l3pool_call(s3[0, 0], s3[0, 1], g3a, s3[1, 0], s3[1, 1], g3b,
                        dinv, W3, b3.reshape(1, H3),
                        batchp, Wfc, bfc.reshape(1, 1))
